# Initial kernel scaffold; baseline (speedup 1.0000x reference)
import jax
import jax.numpy as jnp
from jax.experimental import pallas as pl


def kernel(x, W1, B1, W2, B2, Wsem, bsem, Wgen, bgen):
    raise NotImplementedError("write your pallas kernel here")


# speedup vs baseline: 37.9189x; 37.9189x over previous
import jax
import jax.numpy as jnp
import numpy as np
from jax.experimental import pallas as pl

_NP = 10
_IMG = 256
_HALF = _IMG // 2 + 1
_NBINS = _IMG * _HALF


def _dot(a, b):
    return jax.lax.dot(a, b, precision=jax.lax.Precision.HIGHEST,
                       preferred_element_type=jnp.float32)


def _dft_stats_kernel(x_ref, cm_ref, sm_ref, out_ref):
    x = x_ref[0]
    cm = cm_ref[...]
    sm = sm_ref[...]
    p = _dot(x, cm)
    q = _dot(x, sm)
    fre = _dot(cm, p) - _dot(sm, q)
    fim = -(_dot(cm, q) + _dot(sm, p))
    mag = jnp.sqrt(fre * fre + fim * fim)
    ang = jnp.arctan2(fim, fre)
    col = jax.lax.broadcasted_iota(jnp.int32, (_IMG, _IMG), 1)
    hmask = (col < _HALF).astype(jnp.float32)
    s1 = jnp.sum(mag * hmask)
    s2 = jnp.sum(ang * hmask)
    dc = jnp.sum(x)
    lane = jax.lax.broadcasted_iota(jnp.int32, (1, 128), 1)
    out_ref[0] = jnp.where(
        lane == 0, s1, jnp.where(lane == 1, s2, jnp.where(lane == 2, dc, 0.0)))


def _band_mask(feat, wc1, bc1, wc2, bc2):
    c1 = jax.nn.sigmoid(feat * wc1 + bc1)
    c2 = jax.nn.sigmoid(feat * wc2 + bc2)
    ind = (jnp.clip(c2 - c1, 0.0, None) > 0).astype(jnp.float32)
    c1s = c1 * ind
    cx1 = jnp.clip(jnp.floor(c1s * _IMG), 0.0, _IMG - 1.0)
    return (cx1 == 0.0).astype(jnp.float32)


def _epilogue_kernel(stats_ref,
                     w1xc1, b1xc1, w1xc2, b1xc2,
                     w1yc1, b1yc1, w1yc2, b1yc2,
                     w2xc1, b2xc1, w2xc2, b2xc2,
                     w2yc1, b2yc1, w2yc2, b2yc2,
                     wsem_ref, bsem_ref, wgen_ref, bgen_ref, out_ref):
    stats = stats_ref[...]
    lane = jax.lax.broadcasted_iota(jnp.int32, stats.shape, 1)
    feat1 = jnp.sum(jnp.where(lane == 0, stats, 0.0), axis=1,
                    keepdims=True) / float(_NBINS)
    feat2 = jnp.sum(jnp.where(lane == 1, stats, 0.0), axis=1,
                    keepdims=True) / float(_NBINS)
    dc = jnp.sum(jnp.where(lane == 2, stats, 0.0), axis=1, keepdims=True)

    mask1 = jnp.clip(
        _band_mask(feat1, w1xc1[...], b1xc1[...], w1xc2[...], b1xc2[...]) +
        _band_mask(feat1, w1yc1[...], b1yc1[...], w1yc2[...], b1yc2[...]),
        0.0, 1.0)
    mask2 = jnp.clip(
        _band_mask(feat2, w2xc1[...], b2xc1[...], w2xc2[...], b2xc2[...]) +
        _band_mask(feat2, w2yc1[...], b2yc1[...], w2yc2[...], b2yc2[...]),
        0.0, 1.0)

    c1_00 = jnp.abs(dc)
    c2_00 = jnp.where(dc < 0, jnp.float32(np.pi), 0.0)
    inv = jnp.float32(1.0 / (_IMG * _IMG))
    pd = c1_00 * mask1 * jnp.cos(c2_00 * mask2) * inv
    pc = c1_00 * (1.0 - mask1) * jnp.cos(c2_00 * (1.0 - mask2)) * inv

    rown = jax.lax.broadcasted_iota(jnp.int32, (40, 10), 0)
    coln = jax.lax.broadcasted_iota(jnp.int32, (40, 10), 1)
    nmask = (coln == rown % _NP).astype(jnp.float32)
    selr = jax.lax.broadcasted_iota(jnp.int32, (40, 12), 0)
    selc = jax.lax.broadcasted_iota(jnp.int32, (40, 12), 1)

    def head(pool, w_ref, b_ref):
        w = w_ref[...]
        wrow = jax.lax.broadcasted_iota(jnp.int32, w.shape, 0)
        acc = jnp.zeros((40, 256), jnp.float32) + b_ref[...]
        for c in range(3):
            sel = (selc == (selr // _NP) * 3 + c).astype(jnp.float32)
            g = _dot(sel, pool)
            pcol = jnp.sum(g * nmask, axis=1, keepdims=True)
            wc = jnp.sum(jnp.where(wrow == c, w, 0.0), axis=0, keepdims=True)
            acc = acc + pcol * wc
        return acc

    out_ref[:, 0:256] = head(pc, wsem_ref, bsem_ref)
    out_ref[:, 256:512] = head(pd, wgen_ref, bgen_ref)


def kernel(x, W1, B1, W2, B2, Wsem, bsem, Wgen, bgen):
    B, C, W, H = x.shape
    xi = x.reshape(B * C, W, H)

    idx = jnp.arange(_IMG, dtype=jnp.int32)
    m = (idx[:, None] * idx[None, :]) % _IMG
    theta = (2.0 * np.pi / _IMG) * m.astype(jnp.float32)
    cm = jnp.cos(theta)
    sm = jnp.sin(theta)

    stats = pl.pallas_call(
        _dft_stats_kernel,
        grid=(B * C,),
        in_specs=[
            pl.BlockSpec((1, _IMG, _IMG), lambda i: (i, 0, 0)),
            pl.BlockSpec((_IMG, _IMG), lambda i: (0, 0)),
            pl.BlockSpec((_IMG, _IMG), lambda i: (0, 0)),
        ],
        out_specs=pl.BlockSpec((1, 1, 128), lambda i: (i, 0, 0)),
        out_shape=jax.ShapeDtypeStruct((B * C, 1, 128), jnp.float32),
    )(xi, cm, sm)
    stats = stats.reshape(B * C, 128)

    def wrows(Wm, Bm):
        return [a.reshape(1, _NP) for a in (
            Wm[0, 1], Bm[0, 1], Wm[0, 2], Bm[0, 2],
            Wm[1, 1], Bm[1, 1], Wm[1, 2], Bm[1, 2])]

    args = [stats] + wrows(W1, B1) + wrows(W2, B2) + [
        Wsem, bsem.reshape(1, -1), Wgen, bgen.reshape(1, -1)]

    out = pl.pallas_call(
        _epilogue_kernel,
        out_shape=jax.ShapeDtypeStruct((B * _NP, 512), jnp.float32),
    )(*args)
    return out
